# Initial kernel scaffold; baseline (speedup 1.0000x reference)
import jax
import jax.numpy as jnp
from jax.experimental import pallas as pl


def kernel(x, x_tilde, edge_index, W, prelu_a, Wb):
    raise NotImplementedError("write your pallas kernel here")


# speedup vs baseline: 3.4423x; 3.4423x over previous
import functools

import jax
import jax.numpy as jnp
from jax import lax
from jax.experimental import pallas as pl
from jax.experimental.pallas import tpu as pltpu
from jax.experimental.pallas import tpu_sc as plsc

NC = 2
NS = 16
K = 128
CIB = 32


def _sc_aggregate(x, x_tilde, srcp, dstp, zrows):
    n, d = x.shape
    ns, ng, cib, k = srcp.shape
    zr = zrows.shape[0]
    n_acc = zr * (NS - 1) + 8 * (-(-(n + 1 - zr * (NS - 1)) // 8))
    zl = n_acc - zr * (NS - 1)
    ol = n - zr * (NS - 1)

    mesh = plsc.VectorSubcoreMesh(
        core_axis_name="c", subcore_axis_name="s", num_cores=NC, num_subcores=NS
    )

    @functools.partial(
        pl.kernel,
        out_type=jax.ShapeDtypeStruct((NC, n, d), jnp.float32),
        mesh=mesh,
        scratch_types=[
            pltpu.VMEM((cib, k), jnp.int32),
            pltpu.VMEM((cib, k), jnp.int32),
            pltpu.VMEM((k, d), jnp.float32),
            pltpu.VMEM_SHARED((n_acc, d), jnp.float32),
            pltpu.SemaphoreType.DMA,
        ],
    )
    def agg_kernel(x_hbm, xt_hbm, srcp_hbm, dstp_hbm, z_hbm, out_hbm,
                   src_v, dst_v, rows_v, acc_sh, sem):
        c = lax.axis_index("c")
        s = lax.axis_index("s")

        @pl.when(s < NS - 1)
        def _():
            pltpu.sync_copy(z_hbm, acc_sh.at[pl.ds(s * zr, zr)])

        @pl.when(s == NS - 1)
        def _():
            pltpu.sync_copy(z_hbm.at[pl.ds(0, zl)],
                            acc_sh.at[pl.ds((NS - 1) * zr, zl)])

        plsc.subcore_barrier()

        def run(table_hbm, out2):
            def group(g, carry):
                pltpu.sync_copy(srcp_hbm.at[s].at[g], src_v)
                pltpu.sync_copy(dstp_hbm.at[s].at[g], dst_v)

                def step(i, carry2):
                    pltpu.async_copy(table_hbm.at[src_v.at[i]], rows_v,
                                     sem).wait()
                    pltpu.sync_copy(rows_v, acc_sh.at[dst_v.at[i]], add=True)
                    return carry2

                return lax.fori_loop(0, cib, step, carry)

            lax.fori_loop(0, ng, group, 0)
            plsc.subcore_barrier()

            @pl.when(s < NS - 1)
            def _():
                pltpu.sync_copy(acc_sh.at[pl.ds(s * zr, zr)],
                                out2.at[pl.ds(s * zr, zr)])

            @pl.when(s == NS - 1)
            def _():
                pltpu.sync_copy(acc_sh.at[pl.ds((NS - 1) * zr, ol)],
                                out2.at[pl.ds((NS - 1) * zr, ol)])

        @pl.when(c == 0)
        def _():
            run(x_hbm, out_hbm.at[0])

        @pl.when(c == 1)
        def _():
            run(xt_hbm, out_hbm.at[1])

    return agg_kernel(x, x_tilde, srcp, dstp, zrows)


def _tc_head(agg, w, a11, wb):
    n = agg.shape[1]

    def head_kernel(agg_ref, w_ref, a_ref, wb_ref, out_ref):
        wt = w_ref[...].T
        a = a_ref[0, 0]
        h0 = jnp.dot(agg_ref[0], wt, preferred_element_type=jnp.float32)
        h = jnp.where(h0 >= 0, h0, a * h0)
        g0 = jnp.dot(agg_ref[1], wt, preferred_element_type=jnp.float32)
        g = jnp.where(g0 >= 0, g0, a * g0)
        s = jax.nn.sigmoid(jnp.sum(h, axis=0, keepdims=True) / n)
        v = jnp.dot(s, wb_ref[...].T, preferred_element_type=jnp.float32)
        dp = jnp.sum(h * v, axis=1, keepdims=True)
        dn = jnp.sum(g * v, axis=1, keepdims=True)
        out_ref[...] = jnp.concatenate([dp, dn], axis=1)

    return pl.pallas_call(
        head_kernel,
        out_shape=jax.ShapeDtypeStruct((n, 2), jnp.float32),
    )(agg, w, a11, wb)


def kernel(x, x_tilde, edge_index, W, prelu_a, Wb):
    n, d = x.shape
    e = edge_index.shape[1]
    src = edge_index[0].astype(jnp.int32)
    dst = edge_index[1].astype(jnp.int32)

    ng = -(-e // (NS * CIB * K))
    e_pad = NS * ng * CIB * K
    pad = e_pad - e
    srcp = jnp.concatenate(
        [src, jnp.zeros((pad,), jnp.int32)]).reshape(NS, ng, CIB, K)
    dstp = jnp.concatenate(
        [dst, jnp.full((pad,), n, jnp.int32)]).reshape(NS, ng, CIB, K)
    zr = 8 * (-(-(n + 1) // (8 * NS)))
    zrows = jnp.zeros((zr, d), jnp.float32)

    agg = _sc_aggregate(x, x_tilde, srcp, dstp, zrows)
    out2 = _tc_head(agg, W, jnp.reshape(prelu_a, (1, 1)), Wb[0])
    return out2.T.reshape(1, 2 * n)


# speedup vs baseline: 3.9058x; 1.1347x over previous
import functools

import jax
import jax.numpy as jnp
from jax import lax
from jax.experimental import pallas as pl
from jax.experimental.pallas import tpu as pltpu
from jax.experimental.pallas import tpu_sc as plsc

NC = 2
NS = 16
K = 128
CIB = 32


def _sc_aggregate(x, x_tilde, srcp, dstp, zrows):
    n, d = x.shape
    ns, ng, cib, k = srcp.shape
    zr = zrows.shape[0]
    n_acc = zr * (NS - 1) + 8 * (-(-(n + 1 - zr * (NS - 1)) // 8))
    zl = n_acc - zr * (NS - 1)
    ol = n - zr * (NS - 1)

    mesh = plsc.VectorSubcoreMesh(
        core_axis_name="c", subcore_axis_name="s", num_cores=NC, num_subcores=NS
    )

    @functools.partial(
        pl.kernel,
        out_type=jax.ShapeDtypeStruct((NC, n, d), jnp.float32),
        mesh=mesh,
        scratch_types=[
            pltpu.VMEM((cib, k), jnp.int32),
            pltpu.VMEM((cib, k), jnp.int32),
            pltpu.VMEM((k, d), jnp.float32),
            pltpu.VMEM((k, d), jnp.float32),
            pltpu.VMEM_SHARED((n_acc, d), jnp.float32),
            pltpu.SemaphoreType.DMA,
            pltpu.SemaphoreType.DMA,
        ],
    )
    def agg_kernel(x_hbm, xt_hbm, srcp_hbm, dstp_hbm, z_hbm, out_hbm,
                   src_v, dst_v, rows_a, rows_b, acc_sh, sem_a, sem_b):
        c = lax.axis_index("c")
        s = lax.axis_index("s")

        @pl.when(s < NS - 1)
        def _():
            pltpu.sync_copy(z_hbm, acc_sh.at[pl.ds(s * zr, zr)])

        @pl.when(s == NS - 1)
        def _():
            pltpu.sync_copy(z_hbm.at[pl.ds(0, zl)],
                            acc_sh.at[pl.ds((NS - 1) * zr, zl)])

        plsc.subcore_barrier()

        def run(table_hbm, out2):
            def group(g, carry):
                pltpu.sync_copy(srcp_hbm.at[s].at[g], src_v)
                pltpu.sync_copy(dstp_hbm.at[s].at[g], dst_v)
                pltpu.async_copy(table_hbm.at[src_v.at[0]], rows_a, sem_a)

                def step2(j, carry2):
                    i0 = 2 * j
                    pltpu.make_async_copy(
                        table_hbm.at[src_v.at[i0]], rows_a, sem_a).wait()
                    pltpu.async_copy(
                        table_hbm.at[src_v.at[i0 + 1]], rows_b, sem_b)
                    pltpu.sync_copy(rows_a, acc_sh.at[dst_v.at[i0]], add=True)
                    pltpu.make_async_copy(
                        table_hbm.at[src_v.at[i0 + 1]], rows_b, sem_b).wait()

                    @pl.when(j < cib // 2 - 1)
                    def _():
                        pltpu.async_copy(
                            table_hbm.at[src_v.at[i0 + 2]], rows_a, sem_a)

                    pltpu.sync_copy(rows_b, acc_sh.at[dst_v.at[i0 + 1]],
                                    add=True)
                    return carry2

                return lax.fori_loop(0, cib // 2, step2, carry)

            lax.fori_loop(0, ng, group, 0)
            plsc.subcore_barrier()

            @pl.when(s < NS - 1)
            def _():
                pltpu.sync_copy(acc_sh.at[pl.ds(s * zr, zr)],
                                out2.at[pl.ds(s * zr, zr)])

            @pl.when(s == NS - 1)
            def _():
                pltpu.sync_copy(acc_sh.at[pl.ds((NS - 1) * zr, ol)],
                                out2.at[pl.ds((NS - 1) * zr, ol)])

        @pl.when(c == 0)
        def _():
            run(x_hbm, out_hbm.at[0])

        @pl.when(c == 1)
        def _():
            run(xt_hbm, out_hbm.at[1])

    return agg_kernel(x, x_tilde, srcp, dstp, zrows)


def _tc_head(agg, w, a11, wb):
    n = agg.shape[1]

    def head_kernel(agg_ref, w_ref, a_ref, wb_ref, out_ref):
        wt = w_ref[...].T
        a = a_ref[0, 0]
        h0 = jnp.dot(agg_ref[0], wt, preferred_element_type=jnp.float32)
        h = jnp.where(h0 >= 0, h0, a * h0)
        g0 = jnp.dot(agg_ref[1], wt, preferred_element_type=jnp.float32)
        g = jnp.where(g0 >= 0, g0, a * g0)
        s = jax.nn.sigmoid(jnp.sum(h, axis=0, keepdims=True) / n)
        v = jnp.dot(s, wb_ref[...].T, preferred_element_type=jnp.float32)
        dp = jnp.sum(h * v, axis=1, keepdims=True)
        dn = jnp.sum(g * v, axis=1, keepdims=True)
        out_ref[...] = jnp.concatenate([dp, dn], axis=1)

    return pl.pallas_call(
        head_kernel,
        out_shape=jax.ShapeDtypeStruct((n, 2), jnp.float32),
    )(agg, w, a11, wb)


def kernel(x, x_tilde, edge_index, W, prelu_a, Wb):
    n, d = x.shape
    e = edge_index.shape[1]
    src = edge_index[0].astype(jnp.int32)
    dst = edge_index[1].astype(jnp.int32)

    ng = -(-e // (NS * CIB * K))
    e_pad = NS * ng * CIB * K
    pad = e_pad - e
    srcp = jnp.concatenate(
        [src, jnp.zeros((pad,), jnp.int32)]).reshape(NS, ng, CIB, K)
    dstp = jnp.concatenate(
        [dst, jnp.full((pad,), n, jnp.int32)]).reshape(NS, ng, CIB, K)
    zr = 8 * (-(-(n + 1) // (8 * NS)))
    zrows = jnp.zeros((zr, d), jnp.float32)

    agg = _sc_aggregate(x, x_tilde, srcp, dstp, zrows)
    out2 = _tc_head(agg, W, jnp.reshape(prelu_a, (1, 1)), Wb[0])
    return out2.T.reshape(1, 2 * n)


# speedup vs baseline: 4.0207x; 1.0294x over previous
import functools

import jax
import jax.numpy as jnp
from jax import lax
from jax.experimental import pallas as pl
from jax.experimental.pallas import tpu as pltpu
from jax.experimental.pallas import tpu_sc as plsc

NC = 2
NS = 16
K = 64
CIB = 64
NB = 4
LA = 2


def _sc_aggregate(x, x_tilde, srcp, dstp, zrows):
    n, d = x.shape
    ns, ng, cib, k = srcp.shape
    zr = zrows.shape[0]
    n_acc = zr * (NS - 1) + 8 * (-(-(n + 1 - zr * (NS - 1)) // 8))
    zl = n_acc - zr * (NS - 1)
    ol = n - zr * (NS - 1)

    mesh = plsc.VectorSubcoreMesh(
        core_axis_name="c", subcore_axis_name="s", num_cores=NC, num_subcores=NS
    )

    @functools.partial(
        pl.kernel,
        out_type=jax.ShapeDtypeStruct((NC, n, d), jnp.float32),
        mesh=mesh,
        scratch_types=(
            [pltpu.VMEM((cib, k), jnp.int32)] * 2
            + [pltpu.VMEM((k, d), jnp.float32)] * NB
            + [pltpu.VMEM_SHARED((n_acc, d), jnp.float32)]
            + [pltpu.SemaphoreType.DMA] * (2 * NB)
        ),
    )
    def agg_kernel(x_hbm, xt_hbm, srcp_hbm, dstp_hbm, z_hbm, out_hbm,
                   src_v, dst_v, *rest):
        rows = rest[:NB]
        acc_sh = rest[NB]
        gsem = rest[NB + 1:NB + 1 + NB]
        ssem = rest[NB + 1 + NB:]
        c = lax.axis_index("c")
        s = lax.axis_index("s")

        @pl.when(s < NS - 1)
        def _():
            pltpu.sync_copy(z_hbm, acc_sh.at[pl.ds(s * zr, zr)])

        @pl.when(s == NS - 1)
        def _():
            pltpu.sync_copy(z_hbm.at[pl.ds(0, zl)],
                            acc_sh.at[pl.ds((NS - 1) * zr, zl)])

        plsc.subcore_barrier()

        def run(table_hbm, out2):
            def gstart(i, b):
                pltpu.async_copy(table_hbm.at[src_v.at[i]], rows[b], gsem[b])

            def gwait(i, b):
                pltpu.make_async_copy(
                    table_hbm.at[src_v.at[i]], rows[b], gsem[b]).wait()

            def sstart(i, b):
                pltpu.async_copy(rows[b], acc_sh.at[dst_v.at[i]], ssem[b],
                                 add=True)

            def swait(i, b):
                pltpu.make_async_copy(
                    rows[b], acc_sh.at[dst_v.at[i]], ssem[b]).wait()

            def group(g, carry):
                pltpu.sync_copy(srcp_hbm.at[s].at[g], src_v)
                pltpu.sync_copy(dstp_hbm.at[s].at[g], dst_v)
                for i in range(LA):
                    gstart(i, i % NB)

                def step(j, carry2):
                    for b in range(NB):
                        i = NB * j + b
                        gwait(i, b)
                        sstart(i, b)
                        bn = (b + LA) % NB
                        if b + LA < NB:
                            @pl.when(j >= 1)
                            def _():
                                swait(NB * (j - 1) + bn, bn)

                            gstart(i + LA, bn)
                        else:
                            @pl.when(j < cib // NB - 1)
                            def _():
                                swait(NB * j + bn, bn)
                                gstart(i + LA, bn)
                    return carry2

                lax.fori_loop(0, cib // NB, step, carry)
                for i in range(cib - NB, cib):
                    swait(i, i % NB)
                return carry

            lax.fori_loop(0, ng, group, 0)
            plsc.subcore_barrier()

            @pl.when(s < NS - 1)
            def _():
                pltpu.sync_copy(acc_sh.at[pl.ds(s * zr, zr)],
                                out2.at[pl.ds(s * zr, zr)])

            @pl.when(s == NS - 1)
            def _():
                pltpu.sync_copy(acc_sh.at[pl.ds((NS - 1) * zr, ol)],
                                out2.at[pl.ds((NS - 1) * zr, ol)])

        @pl.when(c == 0)
        def _():
            run(x_hbm, out_hbm.at[0])

        @pl.when(c == 1)
        def _():
            run(xt_hbm, out_hbm.at[1])

    return agg_kernel(x, x_tilde, srcp, dstp, zrows)


def _tc_head(agg, w, a11, wb):
    n = agg.shape[1]

    def head_kernel(agg_ref, w_ref, a_ref, wb_ref, out_ref):
        wt = w_ref[...].T
        a = a_ref[0, 0]
        h0 = jnp.dot(agg_ref[0], wt, preferred_element_type=jnp.float32)
        h = jnp.where(h0 >= 0, h0, a * h0)
        g0 = jnp.dot(agg_ref[1], wt, preferred_element_type=jnp.float32)
        g = jnp.where(g0 >= 0, g0, a * g0)
        s = jax.nn.sigmoid(jnp.sum(h, axis=0, keepdims=True) / n)
        v = jnp.dot(s, wb_ref[...].T, preferred_element_type=jnp.float32)
        dp = jnp.sum(h * v, axis=1, keepdims=True)
        dn = jnp.sum(g * v, axis=1, keepdims=True)
        out_ref[...] = jnp.concatenate([dp, dn], axis=1)

    return pl.pallas_call(
        head_kernel,
        out_shape=jax.ShapeDtypeStruct((n, 2), jnp.float32),
    )(agg, w, a11, wb)


def kernel(x, x_tilde, edge_index, W, prelu_a, Wb):
    n, d = x.shape
    e = edge_index.shape[1]
    src = edge_index[0].astype(jnp.int32)
    dst = edge_index[1].astype(jnp.int32)

    ng = -(-e // (NS * CIB * K))
    e_pad = NS * ng * CIB * K
    pad = e_pad - e
    srcp = jnp.concatenate(
        [src, jnp.zeros((pad,), jnp.int32)]).reshape(NS, ng, CIB, K)
    dstp = jnp.concatenate(
        [dst, jnp.full((pad,), n, jnp.int32)]).reshape(NS, ng, CIB, K)
    zr = 8 * (-(-(n + 1) // (8 * NS)))
    zrows = jnp.zeros((zr, d), jnp.float32)

    agg = _sc_aggregate(x, x_tilde, srcp, dstp, zrows)
    out2 = _tc_head(agg, W, jnp.reshape(prelu_a, (1, 1)), Wb[0])
    return out2.T.reshape(1, 2 * n)


# speedup vs baseline: 4.1459x; 1.0311x over previous
import functools

import jax
import jax.numpy as jnp
from jax import lax
from jax.experimental import pallas as pl
from jax.experimental.pallas import tpu as pltpu
from jax.experimental.pallas import tpu_sc as plsc

NC = 2
NS = 16
K = 64
CIB = 64
NB = 4
LA = 3


def _sc_aggregate(x, x_tilde, srcp, dstp, zrows):
    n, d = x.shape
    ns, ng, cib, k = srcp.shape
    zr = zrows.shape[0]
    n_acc = zr * (NS - 1) + 8 * (-(-(n + 1 - zr * (NS - 1)) // 8))
    zl = n_acc - zr * (NS - 1)
    ol = n - zr * (NS - 1)

    mesh = plsc.VectorSubcoreMesh(
        core_axis_name="c", subcore_axis_name="s", num_cores=NC, num_subcores=NS
    )

    @functools.partial(
        pl.kernel,
        out_type=jax.ShapeDtypeStruct((NC, n, d), jnp.float32),
        mesh=mesh,
        scratch_types=(
            [pltpu.VMEM((cib, k), jnp.int32)] * 2
            + [pltpu.VMEM((k, d), jnp.float32)] * NB
            + [pltpu.VMEM_SHARED((n_acc, d), jnp.float32)]
            + [pltpu.SemaphoreType.DMA] * (2 * NB)
        ),
    )
    def agg_kernel(x_hbm, xt_hbm, srcp_hbm, dstp_hbm, z_hbm, out_hbm,
                   src_v, dst_v, *rest):
        rows = rest[:NB]
        acc_sh = rest[NB]
        gsem = rest[NB + 1:NB + 1 + NB]
        ssem = rest[NB + 1 + NB:]
        c = lax.axis_index("c")
        s = lax.axis_index("s")

        @pl.when(s < NS - 1)
        def _():
            pltpu.sync_copy(z_hbm, acc_sh.at[pl.ds(s * zr, zr)])

        @pl.when(s == NS - 1)
        def _():
            pltpu.sync_copy(z_hbm.at[pl.ds(0, zl)],
                            acc_sh.at[pl.ds((NS - 1) * zr, zl)])

        plsc.subcore_barrier()

        def run(table_hbm, out2):
            def gstart(i, b):
                pltpu.async_copy(table_hbm.at[src_v.at[i]], rows[b], gsem[b])

            def gwait(i, b):
                pltpu.make_async_copy(
                    table_hbm.at[src_v.at[i]], rows[b], gsem[b]).wait()

            def sstart(i, b):
                pltpu.async_copy(rows[b], acc_sh.at[dst_v.at[i]], ssem[b],
                                 add=True)

            def swait(i, b):
                pltpu.make_async_copy(
                    rows[b], acc_sh.at[dst_v.at[i]], ssem[b]).wait()

            def group(g, carry):
                pltpu.sync_copy(srcp_hbm.at[s].at[g], src_v)
                pltpu.sync_copy(dstp_hbm.at[s].at[g], dst_v)
                for i in range(LA):
                    gstart(i, i % NB)

                def step(j, carry2):
                    for b in range(NB):
                        i = NB * j + b
                        gwait(i, b)
                        sstart(i, b)
                        bn = (b + LA) % NB
                        if b + LA < NB:
                            @pl.when(j >= 1)
                            def _():
                                swait(NB * (j - 1) + bn, bn)

                            gstart(i + LA, bn)
                        else:
                            @pl.when(j < cib // NB - 1)
                            def _():
                                swait(NB * j + bn, bn)
                                gstart(i + LA, bn)
                    return carry2

                lax.fori_loop(0, cib // NB, step, carry)
                for i in range(cib - NB, cib):
                    swait(i, i % NB)
                return carry

            lax.fori_loop(0, ng, group, 0)
            plsc.subcore_barrier()

            @pl.when(s < NS - 1)
            def _():
                pltpu.sync_copy(acc_sh.at[pl.ds(s * zr, zr)],
                                out2.at[pl.ds(s * zr, zr)])

            @pl.when(s == NS - 1)
            def _():
                pltpu.sync_copy(acc_sh.at[pl.ds((NS - 1) * zr, ol)],
                                out2.at[pl.ds((NS - 1) * zr, ol)])

        @pl.when(c == 0)
        def _():
            run(x_hbm, out_hbm.at[0])

        @pl.when(c == 1)
        def _():
            run(xt_hbm, out_hbm.at[1])

    return agg_kernel(x, x_tilde, srcp, dstp, zrows)


def _tc_head(agg, w, a11, wb):
    n = agg.shape[1]

    def head_kernel(agg_ref, w_ref, a_ref, wb_ref, out_ref):
        wt = w_ref[...].T
        a = a_ref[0, 0]
        h0 = jnp.dot(agg_ref[0], wt, preferred_element_type=jnp.float32)
        h = jnp.where(h0 >= 0, h0, a * h0)
        g0 = jnp.dot(agg_ref[1], wt, preferred_element_type=jnp.float32)
        g = jnp.where(g0 >= 0, g0, a * g0)
        s = jax.nn.sigmoid(jnp.sum(h, axis=0, keepdims=True) / n)
        v = jnp.dot(s, wb_ref[...].T, preferred_element_type=jnp.float32)
        dp = jnp.sum(h * v, axis=1, keepdims=True)
        dn = jnp.sum(g * v, axis=1, keepdims=True)
        out_ref[...] = jnp.concatenate([dp, dn], axis=1)

    return pl.pallas_call(
        head_kernel,
        out_shape=jax.ShapeDtypeStruct((n, 2), jnp.float32),
    )(agg, w, a11, wb)


def kernel(x, x_tilde, edge_index, W, prelu_a, Wb):
    n, d = x.shape
    e = edge_index.shape[1]
    src = edge_index[0].astype(jnp.int32)
    dst = edge_index[1].astype(jnp.int32)

    ng = -(-e // (NS * CIB * K))
    e_pad = NS * ng * CIB * K
    pad = e_pad - e
    srcp = jnp.concatenate(
        [src, jnp.zeros((pad,), jnp.int32)]).reshape(NS, ng, CIB, K)
    dstp = jnp.concatenate(
        [dst, jnp.full((pad,), n, jnp.int32)]).reshape(NS, ng, CIB, K)
    zr = 8 * (-(-(n + 1) // (8 * NS)))
    zrows = jnp.zeros((zr, d), jnp.float32)

    agg = _sc_aggregate(x, x_tilde, srcp, dstp, zrows)
    out2 = _tc_head(agg, W, jnp.reshape(prelu_a, (1, 1)), Wb[0])
    return out2.T.reshape(1, 2 * n)
